# baseline (device time: 98833 ns/iter reference)
import jax
import jax.numpy as jnp
from jax import lax
from jax.experimental import pallas as pl
from jax.experimental.pallas import tpu as pltpu

N_DEV = 8
N_SUB = 4


def kernel(x, w_mat, scale_x, scale_w):
    m, k_loc = x.shape
    k_loc2, n = w_mat.shape
    assert k_loc == k_loc2
    m_per = m // N_DEV
    n_half = n // 2
    n_sub = n_half // N_SUB

    def body(x_ref, w_ref, sx_ref, sw_ref, out_ref,
             comm_r, comm_l, send_r, recv_r, send_l, recv_l, ack_sem):
        my = lax.axis_index("i")
        right = lax.rem(my + 1, N_DEV)
        left = lax.rem(my + N_DEV - 1, N_DEV)

        w_bf = w_ref[...].astype(jnp.bfloat16)

        def partial(c, col0):
            xs = x_ref[pl.ds(c * m_per, m_per), :].astype(jnp.bfloat16)
            return lax.dot_general(
                xs, w_bf[:, col0:col0 + n_sub], (((1,), (0,)), ((), ())),
                preferred_element_type=jnp.float32,
            )

        def make_rdma(comm, send, recv, s, q, dst):
            return pltpu.make_async_remote_copy(
                src_ref=comm.at[s, q],
                dst_ref=comm.at[s + 1, q],
                send_sem=send.at[s, q],
                recv_sem=recv.at[s, q],
                device_id=(dst,),
                device_id_type=pl.DeviceIdType.MESH,
            )

        col_r = lambda q: q * n_sub
        col_l = lambda q: n_half + q * n_sub

        rdmas = []

        for q in range(N_SUB):
            comm_r[0, q] = partial(left, col_r(q)).astype(jnp.bfloat16)
            rr = make_rdma(comm_r, send_r, recv_r, 0, q, right)
            rr.start()
            comm_l[0, q] = partial(right, col_l(q)).astype(jnp.bfloat16)
            rl = make_rdma(comm_l, send_l, recv_l, 0, q, left)
            rl.start()
            rdmas += [rr, rl]

        for s in range(N_DEV - 1):
            c_r = lax.rem(my + 2 * N_DEV - 2 - s, N_DEV)
            c_l = lax.rem(my + 2 + s, N_DEV)
            last = s == N_DEV - 2
            parts = [(partial(c_r, col_r(q)), partial(c_l, col_l(q)))
                     for q in range(N_SUB)]
            cur = [(make_rdma(comm_r, send_r, recv_r, s, q, right),
                    make_rdma(comm_l, send_l, recv_l, s, q, left))
                   for q in range(N_SUB)]
            scale = sx_ref[0] * sw_ref[0]
            for q in range(N_SUB):
                rr, rl = cur[q]
                pr, plft = parts[q]
                rr.wait_recv()
                tot = comm_r[s + 1, q].astype(jnp.float32) + pr
                if not last:
                    comm_r[s + 1, q] = tot.astype(jnp.bfloat16)
                    nxt = make_rdma(comm_r, send_r, recv_r, s + 1, q, right)
                    nxt.start()
                    rdmas.append(nxt)
                else:
                    out_ref[:, pl.ds(col_r(q), n_sub)] = (
                        jnp.maximum(tot * scale, 0.0))
                rl.wait_recv()
                tot = comm_l[s + 1, q].astype(jnp.float32) + plft
                if not last:
                    comm_l[s + 1, q] = tot.astype(jnp.bfloat16)
                    nxt = make_rdma(comm_l, send_l, recv_l, s + 1, q, left)
                    nxt.start()
                    rdmas.append(nxt)
                else:
                    out_ref[:, pl.ds(col_l(q), n_sub)] = (
                        jnp.maximum(tot * scale, 0.0))

        for rdma in rdmas:
            rdma.wait_send()

        for nbr in (left, right):
            pl.semaphore_signal(
                ack_sem, inc=1,
                device_id=(nbr,), device_id_type=pl.DeviceIdType.MESH,
            )
        pl.semaphore_wait(ack_sem, 2)

    return pl.pallas_call(
        body,
        out_shape=jax.ShapeDtypeStruct((m_per, n), jnp.float32),
        in_specs=[
            pl.BlockSpec(memory_space=pltpu.VMEM),
            pl.BlockSpec(memory_space=pltpu.VMEM),
            pl.BlockSpec(memory_space=pltpu.SMEM),
            pl.BlockSpec(memory_space=pltpu.SMEM),
        ],
        out_specs=pl.BlockSpec(memory_space=pltpu.VMEM),
        scratch_shapes=[
            pltpu.VMEM((N_DEV, N_SUB, m_per, n_sub), jnp.bfloat16),
            pltpu.VMEM((N_DEV, N_SUB, m_per, n_sub), jnp.bfloat16),
            pltpu.SemaphoreType.DMA((N_DEV - 1, N_SUB)),
            pltpu.SemaphoreType.DMA((N_DEV - 1, N_SUB)),
            pltpu.SemaphoreType.DMA((N_DEV - 1, N_SUB)),
            pltpu.SemaphoreType.DMA((N_DEV - 1, N_SUB)),
            pltpu.SemaphoreType.REGULAR,
        ],
    )(x, w_mat, scale_x, scale_w)


# device time: 76388 ns/iter; 1.2938x vs baseline; 1.2938x over previous
import jax
import jax.numpy as jnp
from jax import lax
from jax.experimental import pallas as pl
from jax.experimental.pallas import tpu as pltpu

N_DEV = 8
N_GRP = 3
MASKS = ((4, 3, 1), (3, 1, 4), (1, 4, 3))
ROFF = (0, 176, 352)
RLEN = (176, 176, 160)


def kernel(x, w_mat, scale_x, scale_w):
    m, k_loc = x.shape
    k_loc2, n = w_mat.shape
    assert k_loc == k_loc2
    m_per = m // N_DEV

    def body(x_ref, w_ref, sx_ref, sw_ref, out_ref,
             acc, w_bf, recv0, recv1, recv2,
             send_sems, recv_sems, ack_sem):
        recv = (recv0, recv1, recv2)
        my = lax.axis_index("i")

        w_bf[...] = w_ref[...].astype(jnp.bfloat16)

        def gemm(j):
            xs = x_ref[pl.ds(j * m_per, m_per), :].astype(jnp.bfloat16)
            return lax.dot_general(
                xs, w_bf[...], (((1,), (0,)), ((), ())),
                preferred_element_type=jnp.float32,
            )

        def send(g, slot, chunk_off, partner_mask):
            j = lax.bitwise_xor(my, chunk_off)
            partner = lax.bitwise_xor(my, partner_mask)
            rdma = pltpu.make_async_remote_copy(
                src_ref=acc.at[j, pl.ds(ROFF[g], RLEN[g]), :],
                dst_ref=recv[g].at[slot],
                send_sem=send_sems.at[g, slot],
                recv_sem=recv_sems.at[g, slot],
                device_id=(partner,),
                device_id_type=pl.DeviceIdType.MESH,
            )
            rdma.start()
            return rdma

        def wait_acc(g, slot, chunk_off):
            rdma = pltpu.make_async_remote_copy(
                src_ref=recv[g].at[slot],
                dst_ref=recv[g].at[slot],
                send_sem=send_sems.at[g, slot],
                recv_sem=recv_sems.at[g, slot],
                device_id=(my,),
                device_id_type=pl.DeviceIdType.MESH,
            )
            rdma.wait_recv()
            j = lax.bitwise_xor(my, chunk_off)
            rows = acc.at[j, pl.ds(ROFF[g], RLEN[g]), :]
            rows[...] = (
                rows[...].astype(jnp.float32)
                + recv[g][slot].astype(jnp.float32)
            ).astype(jnp.bfloat16)

        rdmas = []

        r0_deltas = [(v, v ^ w, 0, w) for (u, v, w) in MASKS]
        sends_of = {}
        for g, (u, v, w) in enumerate(MASKS):
            for slot, d in enumerate(r0_deltas[g]):
                sends_of.setdefault(u ^ d, []).append((g, slot))
        for e in (7, 2, 5, 6, 4, 3, 1):
            j = lax.bitwise_xor(my, e)
            acc[pl.ds(j, 1)] = gemm(j).astype(jnp.bfloat16)[None]
            for g, slot in sends_of[e]:
                u = MASKS[g][0]
                rdmas.append(send(g, slot, e, u))
        acc[pl.ds(my, 1)] = gemm(my).astype(jnp.bfloat16)[None]

        for g, (u, v, w) in enumerate(MASKS):
            wait_acc(g, 0, v)
        for g, (u, v, w) in enumerate(MASKS):
            wait_acc(g, 1, v ^ w)
        for g, (u, v, w) in enumerate(MASKS):
            rdmas.append(send(g, 4, v ^ w, v))
            rdmas.append(send(g, 5, v, v))
        for g, (u, v, w) in enumerate(MASKS):
            wait_acc(g, 2, 0)
        for g, (u, v, w) in enumerate(MASKS):
            wait_acc(g, 3, w)

        for g, (u, v, w) in enumerate(MASKS):
            wait_acc(g, 4, w)
            rdmas.append(send(g, 6, w, w))
        for g, (u, v, w) in enumerate(MASKS):
            wait_acc(g, 5, 0)

        scale = sx_ref[0] * sw_ref[0]
        for g, (u, v, w) in enumerate(MASKS):
            rdma = pltpu.make_async_remote_copy(
                src_ref=recv[g].at[6],
                dst_ref=recv[g].at[6],
                send_sem=send_sems.at[g, 6],
                recv_sem=recv_sems.at[g, 6],
                device_id=(my,),
                device_id_type=pl.DeviceIdType.MESH,
            )
            rdma.wait_recv()
            tot = (
                acc[pl.ds(my, 1), pl.ds(ROFF[g], RLEN[g]), :][0]
                .astype(jnp.float32)
                + recv[g][6].astype(jnp.float32)
            )
            out_ref[pl.ds(ROFF[g], RLEN[g]), :] = jnp.maximum(
                tot * scale, 0.0)

        for rdma in rdmas:
            rdma.wait_send()

        for mask in (1, 3, 4):
            pl.semaphore_signal(
                ack_sem, inc=1,
                device_id=(lax.bitwise_xor(my, mask),),
                device_id_type=pl.DeviceIdType.MESH,
            )
        pl.semaphore_wait(ack_sem, 3)

    return pl.pallas_call(
        body,
        out_shape=jax.ShapeDtypeStruct((m_per, n), jnp.float32),
        in_specs=[
            pl.BlockSpec(memory_space=pltpu.VMEM),
            pl.BlockSpec(memory_space=pltpu.VMEM),
            pl.BlockSpec(memory_space=pltpu.SMEM),
            pl.BlockSpec(memory_space=pltpu.SMEM),
        ],
        out_specs=pl.BlockSpec(memory_space=pltpu.VMEM),
        scratch_shapes=[
            pltpu.VMEM((N_DEV, m_per, n), jnp.bfloat16),
            pltpu.VMEM((k_loc, n), jnp.bfloat16),
            pltpu.VMEM((7, RLEN[0], n), jnp.bfloat16),
            pltpu.VMEM((7, RLEN[1], n), jnp.bfloat16),
            pltpu.VMEM((7, RLEN[2], n), jnp.bfloat16),
            pltpu.SemaphoreType.DMA((N_GRP, 7)),
            pltpu.SemaphoreType.DMA((N_GRP, 7)),
            pltpu.SemaphoreType.REGULAR,
        ],
        compiler_params=pltpu.CompilerParams(
            vmem_limit_bytes=100 * 1024 * 1024,
        ),
    )(x, w_mat, scale_x, scale_w)
